# baseline (speedup 1.0000x reference)
import functools

import jax
import jax.numpy as jnp
from jax import lax
from jax.experimental import pallas as pl
from jax.experimental.pallas import tpu as pltpu
from jax.experimental.pallas import tpu_sc as plsc

N = 10000
E = 320000
F = 128
NC = 2
NS = 16
NW = NC * NS
B = 128
RPW = 80
EPAD = NW * RPW * B
ROWS = EPAD // B
RPT = ROWS // NS
NPAD = 10240
SLICE = NPAD // NS
G = 16
NG_FULL = SLICE // G
NG_LAST = (N - (NS - 1) * SLICE) // G

_MESH = plsc.VectorSubcoreMesh(core_axis_name="c", subcore_axis_name="s")


@functools.partial(
    pl.kernel,
    out_type=jax.ShapeDtypeStruct((NPAD,), jnp.float32),
    mesh=_MESH,
    scratch_types=[
        pltpu.VMEM((RPT, B), jnp.int32),
        pltpu.VMEM((RPT, B), jnp.float32),
        pltpu.VMEM((SLICE,), jnp.float32),
        pltpu.VMEM_SHARED((NPAD,), jnp.float32),
    ],
)
def _deg_kernel(dst_hbm, ew_hbm, deg_hbm, idx_v, val_v, work_v, acc):
    c = lax.axis_index("c")
    s = lax.axis_index("s")
    zero16 = jnp.zeros((16,), jnp.float32)

    def zput(i, _):
        work_v[pl.ds(i * 16, 16)] = zero16
        return _

    lax.fori_loop(0, SLICE // 16, zput, None)
    pltpu.sync_copy(work_v, acc.at[pl.ds(s * SLICE, SLICE)])
    plsc.subcore_barrier()

    pltpu.sync_copy(dst_hbm.at[s], idx_v)
    pltpu.sync_copy(ew_hbm.at[s], val_v)

    def scadd(ci, _):
        pltpu.sync_copy(val_v.at[ci], acc.at[idx_v.at[ci]], add=True)
        return _

    lax.fori_loop(0, RPT, scadd, None)
    plsc.subcore_barrier()

    @pl.when(c == 0)
    def _():
        pltpu.sync_copy(acc.at[pl.ds(s * SLICE, SLICE)],
                        deg_hbm.at[pl.ds(s * SLICE, SLICE)])


@functools.partial(
    pl.kernel,
    out_type=jax.ShapeDtypeStruct((NW, RPW, B), jnp.float32),
    mesh=_MESH,
    scratch_types=[
        pltpu.VMEM((RPW, B), jnp.int32),
        pltpu.VMEM((RPW, B), jnp.int32),
        pltpu.VMEM((RPW, B), jnp.float32),
        pltpu.VMEM((RPW, B), jnp.float32),
        pltpu.VMEM((2, B), jnp.float32),
        pltpu.VMEM((2, B), jnp.float32),
        pltpu.SemaphoreType.DMA,
        pltpu.SemaphoreType.DMA,
    ],
)
def _norm_kernel(src_hbm, dst_hbm, ew_hbm, dis_hbm, norm_hbm, sv, dv, wv,
                 nv, a_v, b_v, sem0, sem1):
    c = lax.axis_index("c")
    s = lax.axis_index("s")
    w = c * NS + s
    pltpu.sync_copy(src_hbm.at[w], sv)
    pltpu.sync_copy(dst_hbm.at[w], dv)
    pltpu.sync_copy(ew_hbm.at[w], wv)

    def gath(r, p, sem):
        pltpu.async_copy(dis_hbm.at[sv.at[r]], a_v.at[p], sem)
        pltpu.async_copy(dis_hbm.at[dv.at[r]], b_v.at[p], sem)

    def wait(r, p, sem):
        pltpu.make_async_copy(dis_hbm.at[sv.at[r]], a_v.at[p], sem).wait()
        pltpu.make_async_copy(dis_hbm.at[dv.at[r]], b_v.at[p], sem).wait()

    def scale(r, p):
        for j in range(B // 16):
            sl = pl.ds(j * 16, 16)
            nv[r, sl] = a_v[p, sl] * wv[r, sl] * b_v[p, sl]

    gath(0, 0, sem0)
    gath(1, 1, sem1)

    def pair(t, _):
        ra = 2 * t
        rb = 2 * t + 1
        wait(ra, 0, sem0)
        scale(ra, 0)

        @pl.when(t < RPW // 2 - 1)
        def _():
            gath(ra + 2, 0, sem0)

        wait(rb, 1, sem1)
        scale(rb, 1)

        @pl.when(t < RPW // 2 - 1)
        def _():
            gath(rb + 2, 1, sem1)

        return _

    lax.fori_loop(0, RPW // 2, pair, None)
    pltpu.sync_copy(nv, norm_hbm.at[w])


@functools.partial(
    pl.kernel,
    out_type=jax.ShapeDtypeStruct((NC, N, F), jnp.float32),
    mesh=_MESH,
    scratch_types=[
        pltpu.VMEM((RPW, B), jnp.int32),
        pltpu.VMEM((2, B), jnp.int32),
        pltpu.VMEM((2, B), jnp.float32),
        pltpu.VMEM((2, B, F), jnp.float32),
        pltpu.VMEM((G, F), jnp.float32),
        pltpu.SemaphoreType.DMA,
        pltpu.SemaphoreType.DMA,
        pltpu.SemaphoreType.DMA,
        pltpu.SemaphoreType.DMA,
        pltpu.SemaphoreType.DMA,
        pltpu.VMEM_SHARED((N, F), jnp.float32),
    ],
)
def _msg_kernel(tab_hbm, src_hbm, dst_hbm, norm_hbm, out_hbm,
                dv, sb, nb, rows2, zb, sem_z, sem_g0, sem_g1, sem_m0,
                sem_m1, acc):
    c = lax.axis_index("c")
    s = lax.axis_index("s")
    w = c * NS + s
    ngroups = jnp.where(s == NS - 1, NG_LAST, NG_FULL)

    zero16 = jnp.zeros((16,), jnp.float32)
    for k in range(G):
        for j in range(F // 16):
            zb[k, pl.ds(j * 16, 16)] = zero16

    def zissue(g, _):
        pltpu.async_copy(zb, acc.at[pl.ds(s * SLICE + g * G, G)], sem_z)
        return _

    def zwait(g, _):
        pltpu.make_async_copy(
            zb, acc.at[pl.ds(s * SLICE + g * G, G)], sem_z).wait()
        return _

    lax.fori_loop(0, ngroups, zissue, None)
    pltpu.sync_copy(dst_hbm.at[w], dv)
    lax.fori_loop(0, ngroups, zwait, None)
    plsc.subcore_barrier()

    def meta_issue(ci, p, sem):
        pltpu.async_copy(src_hbm.at[w, ci], sb.at[p], sem)
        pltpu.async_copy(norm_hbm.at[w, ci], nb.at[p], sem)

    def meta_wait(ci, p, sem):
        pltpu.make_async_copy(src_hbm.at[w, ci], sb.at[p], sem).wait()
        pltpu.make_async_copy(norm_hbm.at[w, ci], nb.at[p], sem).wait()

    def gather_issue(p, sem):
        pltpu.async_copy(tab_hbm.at[sb.at[p]], rows2.at[p], sem)

    def gather_wait(p, sem):
        pltpu.make_async_copy(tab_hbm.at[sb.at[p]], rows2.at[p], sem).wait()

    def scale_scatter(ci, p, prefetch):
        vs = [nb[p, pl.ds(jg * 16, 16)] for jg in range(B // 16)]
        prefetch()
        for jg in range(B // 16):
            for k in range(16):
                sc = vs[jg][k]
                b = jg * 16 + k
                for j in range(F // 16):
                    rows2[p, b, pl.ds(j * 16, 16)] = (
                        rows2[p, b, pl.ds(j * 16, 16)] * sc)
        pltpu.sync_copy(rows2.at[p], acc.at[dv.at[ci]], add=True)

    meta_issue(0, 0, sem_m0)
    meta_wait(0, 0, sem_m0)
    gather_issue(0, sem_g0)
    meta_issue(1, 1, sem_m1)

    def pair(t, _):
        ca = 2 * t
        cb = 2 * t + 1
        not_last = t < RPW // 2 - 1

        def pre0():
            @pl.when(not_last)
            def _():
                meta_issue(ca + 2, 0, sem_m0)

        def pre1():
            @pl.when(not_last)
            def _():
                meta_issue(cb + 2, 1, sem_m1)

        meta_wait(cb, 1, sem_m1)
        gather_issue(1, sem_g1)
        gather_wait(0, sem_g0)
        scale_scatter(ca, 0, pre0)

        @pl.when(not_last)
        def _():
            meta_wait(ca + 2, 0, sem_m0)
            gather_issue(0, sem_g0)

        gather_wait(1, sem_g1)
        scale_scatter(cb, 1, pre1)
        return _

    lax.fori_loop(0, RPW // 2, pair, None)
    plsc.subcore_barrier()

    @pl.when(s < NS - 1)
    def _():
        pltpu.sync_copy(acc.at[pl.ds(s * SLICE, SLICE)],
                        out_hbm.at[c, pl.ds(s * SLICE, SLICE)])

    @pl.when(s == NS - 1)
    def _():
        last = N - (NS - 1) * SLICE
        pltpu.sync_copy(acc.at[pl.ds((NS - 1) * SLICE, last)],
                        out_hbm.at[c, pl.ds((NS - 1) * SLICE, last)])


_RB = 2000


def _dis_tc(deg_raw):
    def body(d_ref, dis_ref, sn_ref):
        deg = d_ref[...] + 1.0
        y = jnp.where(deg > 0.0, lax.rsqrt(jnp.maximum(deg, 1e-12)), 0.0)
        dis_ref[...] = y
        sn_ref[...] = y * y

    return pl.pallas_call(
        body,
        out_shape=(
            jax.ShapeDtypeStruct((NPAD // F, F), jnp.float32),
            jax.ShapeDtypeStruct((NPAD // F, F), jnp.float32),
        ),
    )(deg_raw)


def _xw1_tc(x, W1):
    def body(x_ref, w_ref, o_ref):
        o_ref[...] = jnp.dot(x_ref[...], w_ref[...],
                             preferred_element_type=jnp.float32)

    return pl.pallas_call(
        body,
        grid=(N // _RB,),
        in_specs=[
            pl.BlockSpec((_RB, F), lambda i: (i, 0)),
            pl.BlockSpec((F, F), lambda i: (0, 0)),
        ],
        out_specs=pl.BlockSpec((_RB, F), lambda i: (i, 0)),
        out_shape=jax.ShapeDtypeStruct((N, F), jnp.float32),
    )(x, W1)


def _layer2_tc(P, xw1, sn_col, b1, W2):
    def body(p_ref, x_ref, s_ref, b_ref, w_ref, o_ref):
        h = p_ref[0] + p_ref[1] + x_ref[...] * s_ref[...] + b_ref[...]
        h = jnp.maximum(h, 0.0)
        o_ref[...] = jnp.dot(h, w_ref[...],
                             preferred_element_type=jnp.float32)

    return pl.pallas_call(
        body,
        grid=(N // _RB,),
        in_specs=[
            pl.BlockSpec((NC, _RB, F), lambda i: (0, i, 0)),
            pl.BlockSpec((_RB, F), lambda i: (i, 0)),
            pl.BlockSpec((_RB, 1), lambda i: (i, 0)),
            pl.BlockSpec((F,), lambda i: (0,)),
            pl.BlockSpec((F, F), lambda i: (0, 0)),
        ],
        out_specs=pl.BlockSpec((_RB, F), lambda i: (i, 0)),
        out_shape=jax.ShapeDtypeStruct((N, F), jnp.float32),
    )(P, xw1, sn_col, b1, W2)


def _final_tc(Q, xw2, sn_col, b2):
    def body(q_ref, x_ref, s_ref, b_ref, o_ref):
        o_ref[...] = (q_ref[0] + q_ref[1] + x_ref[...] * s_ref[...]
                      + b_ref[...])

    return pl.pallas_call(
        body,
        grid=(N // _RB,),
        in_specs=[
            pl.BlockSpec((NC, _RB, F), lambda i: (0, i, 0)),
            pl.BlockSpec((_RB, F), lambda i: (i, 0)),
            pl.BlockSpec((_RB, 1), lambda i: (i, 0)),
            pl.BlockSpec((F,), lambda i: (0,)),
        ],
        out_specs=pl.BlockSpec((_RB, F), lambda i: (i, 0)),
        out_shape=jax.ShapeDtypeStruct((N, F), jnp.float32),
    )(Q, xw2, sn_col, b2)


def kernel(x, edge_index, edge_attr, u, batch, W1, b1, W2, b2):
    pad = EPAD - E
    zpad_i = jnp.zeros((pad,), jnp.int32)
    zpad_f = jnp.zeros((pad,), jnp.float32)
    src_f = jnp.concatenate([edge_index[0], zpad_i])
    dst_f = jnp.concatenate([edge_index[1], zpad_i])
    ew_f = jnp.concatenate([edge_attr[:, 0], zpad_f])
    src = src_f.reshape(NW, RPW, B)
    dst = dst_f.reshape(NW, RPW, B)
    ew = ew_f.reshape(NW, RPW, B)
    dst_t = dst_f.reshape(NS, RPT, B)
    ew_t = ew_f.reshape(NS, RPT, B)

    deg_raw = _deg_kernel(dst_t, ew_t)
    dis2d, sn2d = _dis_tc(deg_raw.reshape(NPAD // F, F))
    dis = dis2d.reshape(NPAD)
    sn_col = sn2d.reshape(NPAD)[:N, None]
    norm = _norm_kernel(src, dst, ew, dis)

    xw1 = _xw1_tc(x, W1)
    P = _msg_kernel(xw1, src, dst, norm)
    xw2 = _layer2_tc(P, xw1, sn_col, b1, W2)
    Q = _msg_kernel(xw2, src, dst, norm)
    return _final_tc(Q, xw2, sn_col, b2)


# speedup vs baseline: 1.0153x; 1.0153x over previous
import functools

import jax
import jax.numpy as jnp
from jax import lax
from jax.experimental import pallas as pl
from jax.experimental.pallas import tpu as pltpu
from jax.experimental.pallas import tpu_sc as plsc

N = 10000
E = 320000
F = 128
NC = 2
NS = 16
NW = NC * NS
B = 128
RPW = 80
EPAD = NW * RPW * B
ROWS = EPAD // B
RPT = ROWS // NS
NPAD = 10240
SLICE = NPAD // NS
BLK = 8
NBLK = RPW // BLK
G = 16
NG_FULL = SLICE // G
NG_LAST = (N - (NS - 1) * SLICE) // G

_MESH = plsc.VectorSubcoreMesh(core_axis_name="c", subcore_axis_name="s")


@functools.partial(
    pl.kernel,
    out_type=jax.ShapeDtypeStruct((NPAD,), jnp.float32),
    mesh=_MESH,
    scratch_types=[
        pltpu.VMEM((RPT, B), jnp.int32),
        pltpu.VMEM((RPT, B), jnp.float32),
        pltpu.VMEM((SLICE,), jnp.float32),
        pltpu.VMEM_SHARED((NPAD,), jnp.float32),
    ],
)
def _deg_kernel(dst_hbm, ew_hbm, deg_hbm, idx_v, val_v, work_v, acc):
    c = lax.axis_index("c")
    s = lax.axis_index("s")
    zero16 = jnp.zeros((16,), jnp.float32)

    def zput(i, _):
        work_v[pl.ds(i * 16, 16)] = zero16
        return _

    lax.fori_loop(0, SLICE // 16, zput, None)
    pltpu.sync_copy(work_v, acc.at[pl.ds(s * SLICE, SLICE)])
    plsc.subcore_barrier()

    pltpu.sync_copy(dst_hbm.at[s], idx_v)
    pltpu.sync_copy(ew_hbm.at[s], val_v)

    def scadd(ci, _):
        pltpu.sync_copy(val_v.at[ci], acc.at[idx_v.at[ci]], add=True)
        return _

    lax.fori_loop(0, RPT, scadd, None)
    plsc.subcore_barrier()

    @pl.when(c == 0)
    def _():
        pltpu.sync_copy(acc.at[pl.ds(s * SLICE, SLICE)],
                        deg_hbm.at[pl.ds(s * SLICE, SLICE)])


@functools.partial(
    pl.kernel,
    out_type=jax.ShapeDtypeStruct((NW, RPW, B), jnp.float32),
    mesh=_MESH,
    scratch_types=[
        pltpu.VMEM((RPW, B), jnp.int32),
        pltpu.VMEM((RPW, B), jnp.int32),
        pltpu.VMEM((RPW, B), jnp.float32),
        pltpu.VMEM((RPW, B), jnp.float32),
        pltpu.VMEM((2, B), jnp.float32),
        pltpu.VMEM((2, B), jnp.float32),
        pltpu.SemaphoreType.DMA,
        pltpu.SemaphoreType.DMA,
    ],
)
def _norm_kernel(src_hbm, dst_hbm, ew_hbm, dis_hbm, norm_hbm, sv, dv, wv,
                 nv, a_v, b_v, sem0, sem1):
    c = lax.axis_index("c")
    s = lax.axis_index("s")
    w = c * NS + s
    pltpu.sync_copy(src_hbm.at[w], sv)
    pltpu.sync_copy(dst_hbm.at[w], dv)
    pltpu.sync_copy(ew_hbm.at[w], wv)

    def gath(r, p, sem):
        pltpu.async_copy(dis_hbm.at[sv.at[r]], a_v.at[p], sem)
        pltpu.async_copy(dis_hbm.at[dv.at[r]], b_v.at[p], sem)

    def wait(r, p, sem):
        pltpu.make_async_copy(dis_hbm.at[sv.at[r]], a_v.at[p], sem).wait()
        pltpu.make_async_copy(dis_hbm.at[dv.at[r]], b_v.at[p], sem).wait()

    def scale(r, p):
        for j in range(B // 16):
            sl = pl.ds(j * 16, 16)
            nv[r, sl] = a_v[p, sl] * wv[r, sl] * b_v[p, sl]

    gath(0, 0, sem0)
    gath(1, 1, sem1)

    def pair(t, _):
        ra = 2 * t
        rb = 2 * t + 1
        wait(ra, 0, sem0)
        scale(ra, 0)

        @pl.when(t < RPW // 2 - 1)
        def _():
            gath(ra + 2, 0, sem0)

        wait(rb, 1, sem1)
        scale(rb, 1)

        @pl.when(t < RPW // 2 - 1)
        def _():
            gath(rb + 2, 1, sem1)

        return _

    lax.fori_loop(0, RPW // 2, pair, None)
    pltpu.sync_copy(nv, norm_hbm.at[w])


@functools.partial(
    pl.kernel,
    out_type=jax.ShapeDtypeStruct((NC, N, F), jnp.float32),
    mesh=_MESH,
    scratch_types=[
        pltpu.VMEM((2, BLK * B), jnp.int32),
        pltpu.VMEM((2, BLK * B), jnp.float32),
        pltpu.VMEM((2, BLK, B), jnp.int32),
        pltpu.VMEM((2, B, F), jnp.float32),
        pltpu.VMEM((G, F), jnp.float32),
        pltpu.SemaphoreType.DMA,
        pltpu.SemaphoreType.DMA,
        pltpu.SemaphoreType.DMA,
        pltpu.SemaphoreType.DMA,
        pltpu.SemaphoreType.DMA,
        pltpu.VMEM_SHARED((N, F), jnp.float32),
    ],
)
def _msg_kernel(tab_hbm, srcm_hbm, dstm_hbm, normm_hbm, out_hbm,
                sbB, nbB, dbB, rows2, zb, sem_z, sem_g0, sem_g1, sem_m0,
                sem_m1, acc):
    c = lax.axis_index("c")
    s = lax.axis_index("s")
    w = c * NS + s
    ngroups = jnp.where(s == NS - 1, NG_LAST, NG_FULL)

    zero16 = jnp.zeros((16,), jnp.float32)
    for k in range(G):
        for j in range(F // 16):
            zb[k, pl.ds(j * 16, 16)] = zero16

    def zissue(g, _):
        pltpu.async_copy(zb, acc.at[pl.ds(s * SLICE + g * G, G)], sem_z)
        return _

    def zwait(g, _):
        pltpu.make_async_copy(
            zb, acc.at[pl.ds(s * SLICE + g * G, G)], sem_z).wait()
        return _

    lax.fori_loop(0, ngroups, zissue, None)
    lax.fori_loop(0, ngroups, zwait, None)
    plsc.subcore_barrier()

    sems_m = (sem_m0, sem_m1)
    sems_g = (sem_g0, sem_g1)

    def meta_issue(o, pb):
        pltpu.async_copy(srcm_hbm.at[w, o], sbB.at[pb], sems_m[pb])
        pltpu.async_copy(normm_hbm.at[w, o], nbB.at[pb], sems_m[pb])
        pltpu.async_copy(dstm_hbm.at[w, o], dbB.at[pb], sems_m[pb])

    def meta_wait(o, pb):
        pltpu.make_async_copy(srcm_hbm.at[w, o], sbB.at[pb],
                              sems_m[pb]).wait()
        pltpu.make_async_copy(normm_hbm.at[w, o], nbB.at[pb],
                              sems_m[pb]).wait()
        pltpu.make_async_copy(dstm_hbm.at[w, o], dbB.at[pb],
                              sems_m[pb]).wait()

    def gather_issue(pb, j, p):
        idx = sbB.at[pb, pl.ds(j * B, B)]
        pltpu.async_copy(tab_hbm.at[idx], rows2.at[p], sems_g[p])

    def gather_wait(pb, j, p):
        idx = sbB.at[pb, pl.ds(j * B, B)]
        pltpu.make_async_copy(tab_hbm.at[idx], rows2.at[p],
                              sems_g[p]).wait()

    def scale_scatter(pb, j, p):
        for jg in range(B // 16):
            v = nbB[pb, pl.ds(j * B + jg * 16, 16)]
            for k in range(16):
                sc = v[k]
                b = jg * 16 + k
                for jj in range(F // 16):
                    rows2[p, b, pl.ds(jj * 16, 16)] = (
                        rows2[p, b, pl.ds(jj * 16, 16)] * sc)
        pltpu.sync_copy(rows2.at[p], acc.at[dbB.at[pb, j]], add=True)

    meta_issue(0, 0)
    meta_wait(0, 0)
    gather_issue(0, 0, 0)
    meta_issue(1, 1)

    def block(o, pb):
        def pair(u, _):
            gather_issue(pb, 2 * u + 1, 1)
            gather_wait(pb, 2 * u, 0)
            scale_scatter(pb, 2 * u, 0)

            @pl.when(u < BLK // 2 - 1)
            def _():
                gather_issue(pb, 2 * u + 2, 0)

            @pl.when((u == BLK // 2 - 1) & (o < NBLK - 1))
            def _():
                meta_wait(o + 1, 1 - pb)
                gather_issue(1 - pb, 0, 0)

            gather_wait(pb, 2 * u + 1, 1)
            scale_scatter(pb, 2 * u + 1, 1)

            @pl.when((u == BLK // 2 - 1) & (o < NBLK - 2))
            def _():
                meta_issue(o + 2, pb)

            return _

        lax.fori_loop(0, BLK // 2, pair, None)

    def blockpair(tt, _):
        block(2 * tt, 0)
        block(2 * tt + 1, 1)
        return _

    lax.fori_loop(0, NBLK // 2, blockpair, None)
    plsc.subcore_barrier()

    @pl.when(s < NS - 1)
    def _():
        pltpu.sync_copy(acc.at[pl.ds(s * SLICE, SLICE)],
                        out_hbm.at[c, pl.ds(s * SLICE, SLICE)])

    @pl.when(s == NS - 1)
    def _():
        last = N - (NS - 1) * SLICE
        pltpu.sync_copy(acc.at[pl.ds((NS - 1) * SLICE, last)],
                        out_hbm.at[c, pl.ds((NS - 1) * SLICE, last)])


_RB = 2000


def _dis_tc(deg_raw):
    def body(d_ref, dis_ref, sn_ref):
        deg = d_ref[...] + 1.0
        y = jnp.where(deg > 0.0, lax.rsqrt(jnp.maximum(deg, 1e-12)), 0.0)
        dis_ref[...] = y
        sn_ref[...] = y * y

    return pl.pallas_call(
        body,
        out_shape=(
            jax.ShapeDtypeStruct((NPAD // F, F), jnp.float32),
            jax.ShapeDtypeStruct((NPAD // F, F), jnp.float32),
        ),
    )(deg_raw)


def _xw1_tc(x, W1):
    def body(x_ref, w_ref, o_ref):
        o_ref[...] = jnp.dot(x_ref[...], w_ref[...],
                             preferred_element_type=jnp.float32)

    return pl.pallas_call(
        body,
        grid=(N // _RB,),
        in_specs=[
            pl.BlockSpec((_RB, F), lambda i: (i, 0)),
            pl.BlockSpec((F, F), lambda i: (0, 0)),
        ],
        out_specs=pl.BlockSpec((_RB, F), lambda i: (i, 0)),
        out_shape=jax.ShapeDtypeStruct((N, F), jnp.float32),
    )(x, W1)


def _layer2_tc(P, xw1, sn_col, b1, W2):
    def body(p_ref, x_ref, s_ref, b_ref, w_ref, o_ref):
        h = p_ref[0] + p_ref[1] + x_ref[...] * s_ref[...] + b_ref[...]
        h = jnp.maximum(h, 0.0)
        o_ref[...] = jnp.dot(h, w_ref[...],
                             preferred_element_type=jnp.float32)

    return pl.pallas_call(
        body,
        grid=(N // _RB,),
        in_specs=[
            pl.BlockSpec((NC, _RB, F), lambda i: (0, i, 0)),
            pl.BlockSpec((_RB, F), lambda i: (i, 0)),
            pl.BlockSpec((_RB, 1), lambda i: (i, 0)),
            pl.BlockSpec((F,), lambda i: (0,)),
            pl.BlockSpec((F, F), lambda i: (0, 0)),
        ],
        out_specs=pl.BlockSpec((_RB, F), lambda i: (i, 0)),
        out_shape=jax.ShapeDtypeStruct((N, F), jnp.float32),
    )(P, xw1, sn_col, b1, W2)


def _final_tc(Q, xw2, sn_col, b2):
    def body(q_ref, x_ref, s_ref, b_ref, o_ref):
        o_ref[...] = (q_ref[0] + q_ref[1] + x_ref[...] * s_ref[...]
                      + b_ref[...])

    return pl.pallas_call(
        body,
        grid=(N // _RB,),
        in_specs=[
            pl.BlockSpec((NC, _RB, F), lambda i: (0, i, 0)),
            pl.BlockSpec((_RB, F), lambda i: (i, 0)),
            pl.BlockSpec((_RB, 1), lambda i: (i, 0)),
            pl.BlockSpec((F,), lambda i: (0,)),
        ],
        out_specs=pl.BlockSpec((_RB, F), lambda i: (i, 0)),
        out_shape=jax.ShapeDtypeStruct((N, F), jnp.float32),
    )(Q, xw2, sn_col, b2)


def kernel(x, edge_index, edge_attr, u, batch, W1, b1, W2, b2):
    pad = EPAD - E
    zpad_i = jnp.zeros((pad,), jnp.int32)
    zpad_f = jnp.zeros((pad,), jnp.float32)
    src_f = jnp.concatenate([edge_index[0], zpad_i])
    dst_f = jnp.concatenate([edge_index[1], zpad_i])
    ew_f = jnp.concatenate([edge_attr[:, 0], zpad_f])
    src = src_f.reshape(NW, RPW, B)
    dst = dst_f.reshape(NW, RPW, B)
    ew = ew_f.reshape(NW, RPW, B)
    dst_t = dst_f.reshape(NS, RPT, B)
    ew_t = ew_f.reshape(NS, RPT, B)
    src_m = src_f.reshape(NW, NBLK, BLK * B)
    dst_m = dst_f.reshape(NW, NBLK, BLK, B)

    deg_raw = _deg_kernel(dst_t, ew_t)
    dis2d, sn2d = _dis_tc(deg_raw.reshape(NPAD // F, F))
    dis = dis2d.reshape(NPAD)
    sn_col = sn2d.reshape(NPAD)[:N, None]
    norm = _norm_kernel(src, dst, ew, dis)

    norm_m = norm.reshape(NW, NBLK, BLK * B)

    xw1 = _xw1_tc(x, W1)
    P = _msg_kernel(xw1, src_m, dst_m, norm_m)
    xw2 = _layer2_tc(P, xw1, sn_col, b1, W2)
    Q = _msg_kernel(xw2, src_m, dst_m, norm_m)
    return _final_tc(Q, xw2, sn_col, b2)
